# baseline (speedup 1.0000x reference)
import dataclasses
import functools

import jax
import jax.numpy as jnp
from jax import lax
from jax.experimental import pallas as pl
from jax.experimental.pallas import tpu as pltpu
from jax.experimental.pallas import tpu_sc as plsc

ROWS = 16384
COLS = 1000
K = 200

NUM_CORES = 2
NUM_SUBCORES = 16
NW = NUM_CORES * NUM_SUBCORES
ROWS_PER_WORKER = ROWS // NW

R = 32
NCHUNK = ROWS_PER_WORKER // R

_OFFS = tuple(range(0, K - 16, 16)) + (K - 16,)

_mesh = plsc.VectorSubcoreMesh(core_axis_name="c", subcore_axis_name="s")

_cp = pltpu.CompilerParams()
if "needs_layout_passes" in pltpu.CompilerParams.__dataclass_fields__:
    _cp = dataclasses.replace(_cp, needs_layout_passes=False)


@functools.partial(
    pl.kernel,
    mesh=_mesh,
    compiler_params=_cp,
    out_type=jax.ShapeDtypeStruct((ROWS * K,), jnp.float32),
    scratch_types=[
        pltpu.VMEM((R * COLS,), jnp.float32),
        pltpu.VMEM((R * K,), jnp.int32),
        pltpu.VMEM((R * K,), jnp.float32),
    ],
)
def _gather_dim1(in_hbm, idx_hbm, out_hbm, in_v, idx_v, out_v):
    wid = lax.axis_index("s") * NUM_CORES + lax.axis_index("c")
    row0 = wid * ROWS_PER_WORKER

    @pl.loop(0, NCHUNK)
    def _(c):
        base_row = row0 + c * R
        pltpu.sync_copy(in_hbm.at[pl.ds(base_row * COLS, R * COLS)], in_v)
        pltpu.sync_copy(idx_hbm.at[pl.ds(base_row * K, R * K)], idx_v)

        @pl.loop(0, R)
        def _(r):
            in_base = r * COLS
            out_base = r * K
            for off in _OFFS:
                idx16 = idx_v[pl.ds(out_base + off, 16)]
                vals = plsc.load_gather(in_v, [idx16 + in_base])
                out_v[pl.ds(out_base + off, 16)] = vals

        pltpu.sync_copy(out_v, out_hbm.at[pl.ds(base_row * K, R * K)])


def kernel(input, index):
    idx = index.astype(jnp.int32).reshape(-1)
    out = _gather_dim1(input.reshape(-1), idx)
    return out.reshape(ROWS, K)


# speedup vs baseline: 1.7623x; 1.7623x over previous
import dataclasses
import functools

import jax
import jax.numpy as jnp
from jax import lax
from jax.experimental import pallas as pl
from jax.experimental.pallas import tpu as pltpu
from jax.experimental.pallas import tpu_sc as plsc

ROWS = 16384
COLS = 1000
K = 200

NUM_CORES = 2
NUM_SUBCORES = 16
NW = NUM_CORES * NUM_SUBCORES
ROWS_PER_WORKER = ROWS // NW

R = 32
NCHUNK = ROWS_PER_WORKER // R

_OFFS = tuple(range(0, K - 16, 16)) + (K - 16,)

_mesh = plsc.VectorSubcoreMesh(core_axis_name="c", subcore_axis_name="s")

_cp = pltpu.CompilerParams()
if "needs_layout_passes" in pltpu.CompilerParams.__dataclass_fields__:
    _cp = dataclasses.replace(_cp, needs_layout_passes=False)


@functools.partial(
    pl.kernel,
    mesh=_mesh,
    compiler_params=_cp,
    out_type=jax.ShapeDtypeStruct((ROWS, K), jnp.float32),
    scratch_types=[
        pltpu.VMEM((R, COLS), jnp.float32),
        pltpu.VMEM((R, K), jnp.int32),
        pltpu.VMEM((R, K), jnp.float32),
    ],
)
def _gather_dim1(in_hbm, idx_hbm, out_hbm, in_v, idx_v, out_v):
    wid = lax.axis_index("s") * NUM_CORES + lax.axis_index("c")
    row0 = wid * ROWS_PER_WORKER

    @pl.loop(0, NCHUNK)
    def _(c):
        base_row = row0 + c * R
        pltpu.sync_copy(in_hbm.at[pl.ds(base_row, R)], in_v)
        pltpu.sync_copy(idx_hbm.at[pl.ds(base_row, R)], idx_v)

        @pl.loop(0, R)
        def _(r):
            rsplat = jnp.full((16,), r, jnp.int32)
            for off in _OFFS:
                idx16 = idx_v[r, pl.ds(off, 16)]
                vals = plsc.load_gather(in_v, [rsplat, idx16])
                out_v[r, pl.ds(off, 16)] = vals

        pltpu.sync_copy(out_v, out_hbm.at[pl.ds(base_row, R)])


def kernel(input, index):
    return _gather_dim1(input, index.astype(jnp.int32))
